# baseline (speedup 1.0000x reference)
import functools
import math

import jax
import jax.numpy as jnp
from jax import lax
from jax.experimental import pallas as pl
from jax.experimental.pallas import tpu as pltpu
from jax.experimental.pallas import tpu_sc as plsc

D_MODEL = 768
SCALE = math.sqrt(D_MODEL)
LANES = 16
VECS_PER_ROW = D_MODEL // LANES

NUM_WORKERS = 32
CHUNK = 32
NBUF = 4
PRE = 2


def _make_gather(B):
    b_per_w = B // NUM_WORKERS
    n_chunks = b_per_w // CHUNK
    mesh = plsc.VectorSubcoreMesh(core_axis_name="c", subcore_axis_name="s")

    @functools.partial(
        pl.kernel,
        mesh=mesh,
        out_type=jax.ShapeDtypeStruct((B, D_MODEL), jnp.float32),
        scratch_types=[
            pltpu.VMEM((b_per_w,), jnp.int32),
        ] + [pltpu.VMEM((CHUNK, D_MODEL), jnp.float32)] * NBUF + [
            pltpu.SemaphoreType.DMA,
            pltpu.SemaphoreType.DMA,
        ],
    )
    def gather_kernel(idx_hbm, table_hbm, out_hbm, idx_v, *bufs_and_sems):
        bufs = bufs_and_sems[:NBUF]
        gsem, ssem = bufs_and_sems[NBUF:]
        wid = lax.axis_index("s") * 2 + lax.axis_index("c")
        base = wid * b_per_w
        pltpu.sync_copy(idx_hbm.at[pl.ds(base, b_per_w)], idx_v)

        def start_gather(c, buf):
            return pltpu.async_copy(
                table_hbm.at[idx_v.at[pl.ds(c * CHUNK, CHUNK)]], buf, gsem)

        def start_store(c, buf):
            return pltpu.async_copy(
                buf, out_hbm.at[pl.ds(base + c * CHUNK, CHUNK)], ssem)

        gather_cp = {}
        store_cp = {}
        for c in range(min(PRE, n_chunks)):
            gather_cp[c % NBUF] = start_gather(c, bufs[c % NBUF])
        for c in range(n_chunks):
            b = c % NBUF
            ahead = c + PRE
            if ahead < n_chunks:
                ab = ahead % NBUF
                cp = store_cp.pop(ab, None)
                if cp is not None:
                    cp.wait()
                gather_cp[ab] = start_gather(ahead, bufs[ab])
            gather_cp.pop(b).wait()
            buf = bufs[b]

            @pl.loop(0, CHUNK)
            def _scale(r):
                for j in range(VECS_PER_ROW):
                    sl = pl.ds(j * LANES, LANES)
                    buf[r, sl] = buf[r, sl] * SCALE

            store_cp[b] = start_store(c, buf)
        for b in sorted(store_cp):
            store_cp[b].wait()

    return gather_kernel


def kernel(x, table):
    b, s = x.shape
    idx = x.reshape(-1).astype(jnp.int32)
    out = _make_gather(b * s)(idx, table)
    return out.reshape(b, s, D_MODEL)
